# Initial kernel scaffold; baseline (speedup 1.0000x reference)
import jax
import jax.numpy as jnp
from jax.experimental import pallas as pl


def kernel(input, weight):
    raise NotImplementedError("write your pallas kernel here")


# speedup vs baseline: 1.1130x; 1.1130x over previous
import functools

import jax
import jax.numpy as jnp
from jax import lax
from jax.experimental import pallas as pl
from jax.experimental.pallas import tpu as pltpu
from jax.experimental.pallas import tpu_sc as plsc

NUM_ROWS = 1_000_000
DIM = 32
B = 16384 * 50
NC, NS = 2, 16
NW = NC * NS
BPW = B // NW
CHUNK = 512
NCHUNKS = BPW // CHUNK
NBUF = 2
GROUPS = NCHUNKS // NBUF


def _make_gather():
    mesh = plsc.VectorSubcoreMesh(
        core_axis_name="c", subcore_axis_name="s",
        num_cores=NC, num_subcores=NS)

    @functools.partial(
        pl.kernel,
        out_type=jax.ShapeDtypeStruct((B, DIM), jnp.float32),
        mesh=mesh,
        compiler_params=pltpu.CompilerParams(use_tc_tiling_on_sc=False),
        scratch_types=[
            pltpu.VMEM((BPW,), jnp.int32),
            [pltpu.VMEM((CHUNK, DIM), jnp.float32) for _ in range(NBUF)],
            [pltpu.SemaphoreType.DMA for _ in range(NBUF)],
        ],
    )
    def gather_kernel(idx_hbm, table_hbm, out_hbm, idx_v, rows, gsems):
        wid = lax.axis_index("s") * NC + lax.axis_index("c")
        base = wid * BPW
        pltpu.sync_copy(idx_hbm.at[pl.ds(base, BPW)], idx_v)

        def gather_chunk(j, b):
            src = table_hbm.at[idx_v.at[pl.ds(j * CHUNK, CHUNK)]]
            return pltpu.make_async_copy(src, rows[b], gsems[b])

        for b in range(NBUF):
            gather_chunk(b, b).start()

        def group_body(g, _):
            j0 = g * NBUF
            for b in range(NBUF):
                j = j0 + b
                gather_chunk(j, b).wait()
                pltpu.sync_copy(
                    rows[b], out_hbm.at[pl.ds(base + j * CHUNK, CHUNK)])
                nxt = j + NBUF

                @pl.when(nxt < NCHUNKS)
                def _():
                    gather_chunk(nxt, b).start()
            return ()

        lax.fori_loop(0, GROUPS, group_body, (), unroll=False)

    return gather_kernel


_gather = _make_gather()


@jax.jit
def kernel(input, weight):
    idx = input.reshape(-1).astype(jnp.int32)
    out = _gather(idx, weight)
    return out.reshape(input.shape + (DIM,))
